# baseline (speedup 1.0000x reference)
import jax
import jax.numpy as jnp
from jax import lax
from jax.experimental import pallas as pl
from jax.experimental.pallas import tpu as pltpu
from jax.experimental.pallas import tpu_sc as plsc

N, DEG, D, COORD = 10000, 32, 128, 3
BN = 400

_NW = 32
_CH = 320
_CHL = N - 31 * _CH
_ROW = DEG * COORD


def _sc_coord_body(x_hbm, t_hbm, out_hbm, xbuf, tbuf, obuf):
    wid = lax.axis_index("s") * 2 + lax.axis_index("c")
    lanes = lax.iota(jnp.int32, 16)
    masks = [jnp.where(lanes % 3 == d, 1.0, 0.0).astype(jnp.float32)
             for d in range(3)]

    def chunk(n0, nn):
        pltpu.sync_copy(t_hbm.at[pl.ds(n0 * _ROW, nn * _ROW)],
                        tbuf.at[pl.ds(0, nn * _ROW)])
        pltpu.sync_copy(x_hbm.at[pl.ds(n0 * COORD, nn * COORD)],
                        xbuf.at[pl.ds(0, nn * COORD)])

        def node_step(i, carry):
            base = i * _ROW
            acc = [jnp.zeros((16,), jnp.float32) for _ in range(3)]
            for j in range(_ROW // 16):
                v = tbuf[pl.ds(base + j * 16, 16)]
                v = jnp.clip(v, -1000.0, 1000.0)
                r = j % 3
                for c in range(3):
                    acc[c] = acc[c] + v * masks[(c - r) % 3]
            s = [jnp.full((16,), jnp.sum(acc[c]) * (1.0 / DEG),
                          dtype=jnp.float32) for c in range(3)]
            sv = jnp.where(lanes == 0, s[0], jnp.where(lanes == 1, s[1], s[2]))
            xv = jnp.clip(xbuf[pl.ds(i * COORD, 16)], -1000.0, 1000.0)
            obuf[pl.ds(i * COORD, 16)] = xv + sv
            return carry

        lax.fori_loop(0, nn, node_step, 0)
        pltpu.sync_copy(obuf.at[pl.ds(0, nn * COORD)],
                        out_hbm.at[pl.ds(n0 * COORD, nn * COORD)])

    @pl.when(wid < 31)
    def _():
        chunk(wid * _CH, _CH)

    @pl.when(wid == 31)
    def _():
        chunk(31 * _CH, _CHL)


def _tc_body(hh_ref, e_ref, W1_ref, b1_ref, W2_ref, b2_ref, h_ref):
    ef = jnp.sum(e_ref[...], axis=1)
    hh = hh_ref[...]
    W1 = W1_ref[...]
    h1 = (jnp.dot(hh, W1[:D, :], preferred_element_type=jnp.float32)
          + jnp.dot(ef, W1[D:, :], preferred_element_type=jnp.float32)
          + b1_ref[...])
    h1 = h1 * jax.nn.sigmoid(h1)
    h_ref[...] = (hh
                  + jnp.dot(h1, W2_ref[...], preferred_element_type=jnp.float32)
                  + b2_ref[...])


def kernel(x, hh, trans, edge_feature, W1, b1, W2, b2):
    mesh = plsc.VectorSubcoreMesh(core_axis_name="c", subcore_axis_name="s")
    coord_flat = pl.kernel(
        _sc_coord_body,
        out_type=jax.ShapeDtypeStruct((N * COORD,), jnp.float32),
        mesh=mesh,
        scratch_types=[
            pltpu.VMEM((_CH * COORD + 16,), jnp.float32),
            pltpu.VMEM((_CH * _ROW,), jnp.float32),
            pltpu.VMEM((_CH * COORD + 16,), jnp.float32),
        ],
        compiler_params=pltpu.CompilerParams(needs_layout_passes=False),
    )(x.reshape(-1), trans.reshape(-1))
    coord = coord_flat.reshape(N, COORD)

    b1r = b1.reshape(1, D)
    b2r = b2.reshape(1, D)
    h = pl.pallas_call(
        _tc_body,
        grid=(N // BN,),
        in_specs=[
            pl.BlockSpec((BN, D), lambda i: (i, 0)),
            pl.BlockSpec((BN, DEG, D), lambda i: (i, 0, 0)),
            pl.BlockSpec((2 * D, D), lambda i: (0, 0)),
            pl.BlockSpec((1, D), lambda i: (0, 0)),
            pl.BlockSpec((D, D), lambda i: (0, 0)),
            pl.BlockSpec((1, D), lambda i: (0, 0)),
        ],
        out_specs=pl.BlockSpec((BN, D), lambda i: (i, 0)),
        out_shape=jax.ShapeDtypeStruct((N, D), jnp.float32),
        compiler_params=pltpu.CompilerParams(
            dimension_semantics=("arbitrary",),
        ),
    )(hh, edge_feature, W1, b1r, W2, b2r)
    return coord, h


# speedup vs baseline: 3.5505x; 3.5505x over previous
import jax
import jax.numpy as jnp
from jax import lax
from jax.experimental import pallas as pl
from jax.experimental.pallas import tpu as pltpu

N, DEG, D, COORD = 10000, 32, 128, 3
BN = 400


def _body(x_ref, hh_ref, t_ref, e_ref, W1_ref, b1_ref, W2_ref, b2_ref,
          coord_ref, h_ref):
    t = jnp.clip(t_ref[...], -1000.0, 1000.0)
    j = lax.broadcasted_iota(jnp.int32, (DEG * COORD, COORD), 0)
    c = lax.broadcasted_iota(jnp.int32, (DEG * COORD, COORD), 1)
    S = jnp.where(j % COORD == c, 1.0 / DEG, 0.0).astype(jnp.float32)
    xb = jnp.clip(x_ref[...], -1000.0, 1000.0)
    coord_ref[...] = xb + jnp.dot(t, S, preferred_element_type=jnp.float32)

    ef = jnp.sum(e_ref[...], axis=1)
    hh = hh_ref[...]
    W1 = W1_ref[...]
    h1 = (jnp.dot(hh, W1[:D, :], preferred_element_type=jnp.float32)
          + jnp.dot(ef, W1[D:, :], preferred_element_type=jnp.float32)
          + b1_ref[...])
    h1 = h1 * jax.nn.sigmoid(h1)
    h_ref[...] = (hh
                  + jnp.dot(h1, W2_ref[...], preferred_element_type=jnp.float32)
                  + b2_ref[...])


def kernel(x, hh, trans, edge_feature, W1, b1, W2, b2):
    t2 = trans.reshape(N, DEG * COORD)
    b1r = b1.reshape(1, D)
    b2r = b2.reshape(1, D)
    grid = (N // BN,)
    coord, h = pl.pallas_call(
        _body,
        grid=grid,
        in_specs=[
            pl.BlockSpec((BN, COORD), lambda i: (i, 0)),
            pl.BlockSpec((BN, D), lambda i: (i, 0)),
            pl.BlockSpec((BN, DEG * COORD), lambda i: (i, 0)),
            pl.BlockSpec((BN, DEG, D), lambda i: (i, 0, 0)),
            pl.BlockSpec((2 * D, D), lambda i: (0, 0)),
            pl.BlockSpec((1, D), lambda i: (0, 0)),
            pl.BlockSpec((D, D), lambda i: (0, 0)),
            pl.BlockSpec((1, D), lambda i: (0, 0)),
        ],
        out_specs=[
            pl.BlockSpec((BN, COORD), lambda i: (i, 0)),
            pl.BlockSpec((BN, D), lambda i: (i, 0)),
        ],
        out_shape=[
            jax.ShapeDtypeStruct((N, COORD), jnp.float32),
            jax.ShapeDtypeStruct((N, D), jnp.float32),
        ],
        compiler_params=pltpu.CompilerParams(
            dimension_semantics=("arbitrary",),
        ),
    )(x, hh, t2, edge_feature, W1, b1r, W2, b2r)
    return coord, h
